# baseline (speedup 1.0000x reference)
import functools

import jax
import jax.numpy as jnp
from jax import lax
from jax.experimental import pallas as pl
from jax.experimental.pallas import tpu as pltpu
from jax.experimental.pallas import tpu_sc as plsc

NB = 4096
T = 50
D = 128
V = 1000
NC, NS = 2, 16
NW = NC * NS
BPW = NB // NW
NBUF = 5
NOUTER = T // NBUF

_mesh = plsc.VectorSubcoreMesh(core_axis_name="c", subcore_axis_name="s")


@functools.partial(
    pl.kernel,
    mesh=_mesh,
    out_type=jax.ShapeDtypeStruct((T, NB, D), jnp.float32),
    scratch_types=[
        pltpu.VMEM((T, BPW), jnp.int32),
        pltpu.VMEM((NBUF, BPW, D), jnp.float32),
        pltpu.VMEM_SHARED((V, D), jnp.float32),
        pltpu.SemaphoreType.DMA((NBUF,)),
        pltpu.SemaphoreType.DMA((NBUF,)),
    ],
)
def _gather_kernel(idx_hbm, table_hbm, out_hbm, idx_v, rows_v, table_sp, gsem, osem):
    wid = lax.axis_index("s") * NC + lax.axis_index("c")
    col0 = wid * BPW

    @pl.when(lax.axis_index("s") == 0)
    def _():
        pltpu.sync_copy(table_hbm, table_sp)

    pltpu.sync_copy(idx_hbm.at[:, pl.ds(col0, BPW)], idx_v)
    plsc.subcore_barrier()

    def gather(b, t):
        return pltpu.make_async_copy(
            table_sp.at[idx_v.at[t]], rows_v.at[b], gsem.at[b]
        )

    def out_copy(b, t):
        return pltpu.make_async_copy(
            rows_v.at[b], out_hbm.at[t, pl.ds(col0, BPW)], osem.at[b]
        )

    for b in range(NBUF):
        gather(b, b).start()

    def body(g, carry):
        for b in range(NBUF):
            t = g * NBUF + b
            gather(b, t).wait()
            out_copy(b, t).start()
            out_copy(b, t).wait()
            gather(b, t + NBUF).start()
        return carry

    lax.fori_loop(0, NOUTER - 1, body, 0, unroll=False)

    for b in range(NBUF):
        t = (NOUTER - 1) * NBUF + b
        gather(b, t).wait()
        out_copy(b, t).start()
        out_copy(b, t).wait()


def kernel(prefix_token_ids, prefix_embedding):
    idx_t = prefix_token_ids.T.astype(jnp.int32)
    out_t = _gather_kernel(idx_t, prefix_embedding)
    return out_t.transpose(1, 0, 2)
